# baseline (speedup 1.0000x reference)
import functools

import jax
import jax.numpy as jnp
from jax import lax
from jax.experimental import pallas as pl
from jax.experimental.pallas import tpu as pltpu
from jax.experimental.pallas import tpu_sc as plsc

N_NODES = 10000
D = 128
NUM_GRAPHS = 64
NEG_SLOPE = 0.01

ROW_TILE = 2000
CH = 80
SCH = 25
NW = 32
ACC_ROWS = 10240


def _leaky(x):
    return jnp.where(x >= 0, x, NEG_SLOPE * x)


def _fc1_body(x_ref, w_ref, b_ref, h_ref, r_ref):
    h = x_ref[...]
    W = w_ref[...]
    B = b_ref[...]
    n = W.shape[0]
    for l in range(n):
        h = jnp.dot(h, W[l], preferred_element_type=jnp.float32) + B[l:l + 1, :]
        if l < n - 1:
            h = jnp.maximum(h, 0.0)
    h = _leaky(h)
    h_ref[...] = h
    r_ref[...] = jnp.maximum(h, 0.0)


def _fc1(x, fc1_W, fc1_b):
    n_tiles = x.shape[0] // ROW_TILE
    return pl.pallas_call(
        _fc1_body,
        grid=(n_tiles,),
        in_specs=[
            pl.BlockSpec((ROW_TILE, D), lambda i: (i, 0)),
            pl.BlockSpec(fc1_W.shape, lambda i: (0, 0, 0)),
            pl.BlockSpec(fc1_b.shape, lambda i: (0, 0)),
        ],
        out_specs=[
            pl.BlockSpec((ROW_TILE, D), lambda i: (i, 0)),
            pl.BlockSpec((ROW_TILE, D), lambda i: (i, 0)),
        ],
        out_shape=[
            jax.ShapeDtypeStruct(x.shape, jnp.float32),
            jax.ShapeDtypeStruct(x.shape, jnp.float32),
        ],
    )(x, fc1_W, fc1_b)


def _sc_agg(r, src3d, dst3d):
    nsup = src3d.shape[0] // NW
    zc = ACC_ROWS // 16 // CH
    orows = ACC_ROWS // 16
    mesh = plsc.VectorSubcoreMesh(core_axis_name="c", subcore_axis_name="s")

    @functools.partial(
        pl.kernel,
        mesh=mesh,
        out_type=jax.ShapeDtypeStruct((2, ACC_ROWS, D), jnp.float32),
        scratch_types=[
            pltpu.VMEM((SCH, CH), jnp.int32),
            pltpu.VMEM((SCH, CH), jnp.int32),
            pltpu.VMEM((CH, D), jnp.float32),
            pltpu.VMEM((CH, D), jnp.float32),
            pltpu.VMEM((CH, D), jnp.float32),
            pltpu.VMEM((CH, D), jnp.float32),
            pltpu.VMEM_SHARED((ACC_ROWS, D), jnp.float32),
            pltpu.SemaphoreType.DMA,
            pltpu.SemaphoreType.DMA,
            pltpu.SemaphoreType.DMA,
            pltpu.SemaphoreType.DMA,
        ],
    )
    def k(r_hbm, src_hbm, dst_hbm, out_hbm, src_v, dst_v, rows0, rows1,
          rows2, rows3, acc_sh, sem0, sem1, sem2, sem3):
        cid = lax.axis_index("c")
        sid = lax.axis_index("s")
        wid = cid * 16 + sid
        rows = (rows0, rows1, rows2, rows3)
        sems = (sem0, sem1, sem2, sem3)

        @pl.loop(0, CH)
        def _(rr):
            @pl.loop(0, D, step=16)
            def _(cc):
                rows0.at[pl.ds(rr, 1), pl.ds(cc, 16)][...] = jnp.zeros(
                    (1, 16), jnp.float32)

        @pl.loop(0, zc)
        def _(kk):
            pltpu.sync_copy(rows0,
                            acc_sh.at[pl.ds(sid * (zc * CH) + kk * CH, CH)])

        plsc.subcore_barrier()

        @pl.loop(0, nsup)
        def _(s):
            pltpu.sync_copy(src_hbm.at[wid * nsup + s], src_v)
            pltpu.sync_copy(dst_hbm.at[wid * nsup + s], dst_v)
            for p in range(3):
                pltpu.async_copy(r_hbm.at[src_v.at[p]], rows[p], sems[p])

            @pl.loop(0, SCH, step=4)
            def _(c):
                for kk in range(4):
                    slot = kk
                    pref = (kk + 3) % 4

                    def blk(kk=kk, slot=slot, pref=pref):
                        @pl.when(c + kk + 3 < SCH)
                        def _():
                            pltpu.async_copy(
                                r_hbm.at[src_v.at[c + kk + 3]],
                                rows[pref], sems[pref])

                        pltpu.make_async_copy(r_hbm.at[src_v.at[c + kk]],
                                              rows[slot], sems[slot]).wait()
                        pltpu.sync_copy(rows[slot],
                                        acc_sh.at[dst_v.at[c + kk]],
                                        add=True)

                    if kk == 0:
                        blk()
                    else:
                        pl.when(c + kk < SCH)(blk)

        plsc.subcore_barrier()

        pltpu.sync_copy(acc_sh.at[pl.ds(sid * orows, orows)],
                        out_hbm.at[cid, pl.ds(sid * orows, orows)])

    return k(r, src3d, dst3d)


def _post_body(lo_ref, hi_ref, seg_ref, h_ref, agg_ref, w1_ref, b1_ref,
               w2_ref, b2_ref, fw_ref, fb_ref, out_ref, pooled_ref):
    i = pl.program_id(0)
    z = h_ref[...] + agg_ref[0] + agg_ref[1]
    z = jnp.maximum(jnp.dot(z, w1_ref[...],
                            preferred_element_type=jnp.float32)
                    + b1_ref[...][0:1, :], 0.0)
    z = jnp.dot(z, w2_ref[...],
                preferred_element_type=jnp.float32) + b2_ref[...][0:1, :]
    z = _leaky(z)

    @pl.when(i == 0)
    def _():
        pooled_ref[...] = jnp.full((NUM_GRAPHS, D), -jnp.inf, jnp.float32)

    rowid = lax.broadcasted_iota(jnp.int32, (ROW_TILE, 1), 0) + i * ROW_TILE
    gid = lax.broadcasted_iota(jnp.int32, (NUM_GRAPHS, 1), 0)

    def body(g, pooled):
        m = (rowid >= seg_ref[g]) & (rowid < seg_ref[g + 1])
        loc = jnp.max(jnp.where(m, z, -jnp.inf), axis=0)
        return jnp.where(gid == g, jnp.maximum(pooled, loc[None, :]),
                         pooled)

    pooled_ref[...] = lax.fori_loop(lo_ref[i], hi_ref[i] + 1, body,
                                    pooled_ref[...])

    @pl.when(i == pl.num_programs(0) - 1)
    def _():
        p = pooled_ref[...]
        FW = fw_ref[...]
        FB = fb_ref[...]
        n = FW.shape[0]
        for l in range(n):
            p = jnp.dot(p, FW[l],
                        preferred_element_type=jnp.float32) + FB[l:l + 1, :]
            if l < n - 1:
                p = jnp.maximum(p, 0.0)
        out_ref[...] = p


def _post(h, agg, batch, nn_W1, nn_b1, nn_W2, nn_b2, fc2_W, fc2_b):
    n_tiles = h.shape[0] // ROW_TILE
    b1 = nn_b1.reshape(1, D)
    b2 = nn_b2.reshape(1, D)
    batch = batch.astype(jnp.int32)
    btiles = batch.reshape(n_tiles, ROW_TILE)
    lo = btiles[:, 0]
    hi = btiles[:, -1]
    seg = jnp.sum(batch[None, :] <
                  jnp.arange(NUM_GRAPHS + 1, dtype=jnp.int32)[:, None],
                  axis=1, dtype=jnp.int32)
    return pl.pallas_call(
        _post_body,
        grid=(n_tiles,),
        in_specs=[
            pl.BlockSpec(memory_space=pltpu.SMEM),
            pl.BlockSpec(memory_space=pltpu.SMEM),
            pl.BlockSpec(memory_space=pltpu.SMEM),
            pl.BlockSpec((ROW_TILE, D), lambda i: (i, 0)),
            pl.BlockSpec((2, ROW_TILE, D), lambda i: (0, i, 0)),
            pl.BlockSpec((D, D), lambda i: (0, 0)),
            pl.BlockSpec((1, D), lambda i: (0, 0)),
            pl.BlockSpec((D, D), lambda i: (0, 0)),
            pl.BlockSpec((1, D), lambda i: (0, 0)),
            pl.BlockSpec(fc2_W.shape, lambda i: (0, 0, 0)),
            pl.BlockSpec(fc2_b.shape, lambda i: (0, 0)),
        ],
        out_specs=pl.BlockSpec((NUM_GRAPHS, D), lambda i: (0, 0)),
        out_shape=jax.ShapeDtypeStruct((NUM_GRAPHS, D), jnp.float32),
        scratch_shapes=[pltpu.VMEM((NUM_GRAPHS, D), jnp.float32)],
    )(lo, hi, seg, h, agg, nn_W1, b1, nn_W2, b2, fc2_W, fc2_b)


def kernel(x, edge_index, batch, fc1_W, fc1_b, nn_W1, nn_b1, nn_W2, nn_b2,
           fc2_W, fc2_b):
    E = edge_index.shape[1]
    epw = -(-E // (NW * SCH * CH)) * SCH * CH
    e_pad = epw * NW
    pad = e_pad - E

    src = edge_index[0].astype(jnp.int32)
    dst = edge_index[1].astype(jnp.int32)
    if pad:
        ar = jnp.arange(pad, dtype=jnp.int32)
        src = jnp.concatenate([src, (ar * 97) % N_NODES])
        dst = jnp.concatenate([dst, N_NODES + ar % (ACC_ROWS - N_NODES)])
    src3d = src.reshape(NW * (epw // (SCH * CH)), SCH, CH)
    dst3d = dst.reshape(NW * (epw // (SCH * CH)), SCH, CH)
    h, r = _fc1(x, fc1_W, fc1_b)
    agg = _sc_agg(r, src3d, dst3d)
    return _post(h, agg, batch, nn_W1, nn_b1, nn_W2, nn_b2, fc2_W, fc2_b)


# speedup vs baseline: 1.0264x; 1.0264x over previous
import functools

import jax
import jax.numpy as jnp
from jax import lax
from jax.experimental import pallas as pl
from jax.experimental.pallas import tpu as pltpu
from jax.experimental.pallas import tpu_sc as plsc

N_NODES = 10000
D = 128
NUM_GRAPHS = 64
NEG_SLOPE = 0.01

ROW_TILE = 1000
CH = 80
SCH = 25
NW = 32
ACC_ROWS = 10240


def _leaky(x):
    return jnp.where(x >= 0, x, NEG_SLOPE * x)


def _fc1_body(x_ref, w_ref, b_ref, h_ref, r_ref):
    h = x_ref[...]
    W = w_ref[...]
    B = b_ref[...]
    n = W.shape[0]
    for l in range(n):
        h = jnp.dot(h, W[l], preferred_element_type=jnp.float32) + B[l:l + 1, :]
        if l < n - 1:
            h = jnp.maximum(h, 0.0)
    h = _leaky(h)
    h_ref[...] = h
    r_ref[...] = jnp.maximum(h, 0.0)


def _fc1(x, fc1_W, fc1_b):
    n_tiles = x.shape[0] // ROW_TILE
    return pl.pallas_call(
        _fc1_body,
        grid=(n_tiles,),
        in_specs=[
            pl.BlockSpec((ROW_TILE, D), lambda i: (i, 0)),
            pl.BlockSpec(fc1_W.shape, lambda i: (0, 0, 0)),
            pl.BlockSpec(fc1_b.shape, lambda i: (0, 0)),
        ],
        out_specs=[
            pl.BlockSpec((ROW_TILE, D), lambda i: (i, 0)),
            pl.BlockSpec((ROW_TILE, D), lambda i: (i, 0)),
        ],
        out_shape=[
            jax.ShapeDtypeStruct(x.shape, jnp.float32),
            jax.ShapeDtypeStruct(x.shape, jnp.float32),
        ],
    )(x, fc1_W, fc1_b)


def _sc_agg(r, src3d, dst3d):
    nsup = src3d.shape[0] // NW
    zc = ACC_ROWS // 16 // CH
    orows = ACC_ROWS // 16
    mesh = plsc.VectorSubcoreMesh(core_axis_name="c", subcore_axis_name="s")

    @functools.partial(
        pl.kernel,
        mesh=mesh,
        out_type=jax.ShapeDtypeStruct((2, ACC_ROWS, D), jnp.float32),
        scratch_types=[
            pltpu.VMEM((SCH, CH), jnp.int32),
            pltpu.VMEM((SCH, CH), jnp.int32),
            pltpu.VMEM((CH, D), jnp.float32),
            pltpu.VMEM((CH, D), jnp.float32),
            pltpu.VMEM((CH, D), jnp.float32),
            pltpu.VMEM((CH, D), jnp.float32),
            pltpu.VMEM_SHARED((ACC_ROWS, D), jnp.float32),
            pltpu.SemaphoreType.DMA,
            pltpu.SemaphoreType.DMA,
            pltpu.SemaphoreType.DMA,
            pltpu.SemaphoreType.DMA,
        ],
    )
    def k(r_hbm, src_hbm, dst_hbm, out_hbm, src_v, dst_v, rows0, rows1,
          rows2, rows3, acc_sh, sem0, sem1, sem2, sem3):
        cid = lax.axis_index("c")
        sid = lax.axis_index("s")
        wid = cid * 16 + sid
        rows = (rows0, rows1, rows2, rows3)
        sems = (sem0, sem1, sem2, sem3)

        @pl.loop(0, CH)
        def _(rr):
            @pl.loop(0, D, step=16)
            def _(cc):
                rows0.at[pl.ds(rr, 1), pl.ds(cc, 16)][...] = jnp.zeros(
                    (1, 16), jnp.float32)

        @pl.loop(0, zc)
        def _(kk):
            pltpu.sync_copy(rows0,
                            acc_sh.at[pl.ds(sid * (zc * CH) + kk * CH, CH)])

        plsc.subcore_barrier()

        @pl.loop(0, nsup)
        def _(s):
            pltpu.sync_copy(src_hbm.at[wid * nsup + s], src_v)
            pltpu.sync_copy(dst_hbm.at[wid * nsup + s], dst_v)
            for p in range(3):
                pltpu.async_copy(r_hbm.at[src_v.at[p]], rows[p], sems[p])

            @pl.loop(0, SCH, step=4)
            def _(c):
                for kk in range(4):
                    slot = kk
                    pref = (kk + 3) % 4

                    def blk(kk=kk, slot=slot, pref=pref):
                        @pl.when(c + kk + 3 < SCH)
                        def _():
                            pltpu.async_copy(
                                r_hbm.at[src_v.at[c + kk + 3]],
                                rows[pref], sems[pref])

                        pltpu.make_async_copy(r_hbm.at[src_v.at[c + kk]],
                                              rows[slot], sems[slot]).wait()
                        pltpu.sync_copy(rows[slot],
                                        acc_sh.at[dst_v.at[c + kk]],
                                        add=True)

                    if kk == 0:
                        blk()
                    else:
                        pl.when(c + kk < SCH)(blk)

        plsc.subcore_barrier()

        pltpu.sync_copy(acc_sh.at[pl.ds(sid * orows, orows)],
                        out_hbm.at[cid, pl.ds(sid * orows, orows)])

    return k(r, src3d, dst3d)


def _post_body(lo_ref, hi_ref, seg_ref, h_ref, agg_ref, w1_ref, b1_ref,
               w2_ref, b2_ref, fw_ref, fb_ref, out_ref, pooled_ref):
    i = pl.program_id(0)
    z = h_ref[...] + agg_ref[0] + agg_ref[1]
    z = jnp.maximum(jnp.dot(z, w1_ref[...],
                            preferred_element_type=jnp.float32)
                    + b1_ref[...][0:1, :], 0.0)
    z = jnp.dot(z, w2_ref[...],
                preferred_element_type=jnp.float32) + b2_ref[...][0:1, :]
    z = _leaky(z)

    @pl.when(i == 0)
    def _():
        pooled_ref[...] = jnp.full((NUM_GRAPHS, D), -jnp.inf, jnp.float32)

    rowid = lax.broadcasted_iota(jnp.int32, (ROW_TILE, 1), 0) + i * ROW_TILE
    gid = lax.broadcasted_iota(jnp.int32, (NUM_GRAPHS, 1), 0)

    def body(g, pooled):
        m = (rowid >= seg_ref[g]) & (rowid < seg_ref[g + 1])
        loc = jnp.max(jnp.where(m, z, -jnp.inf), axis=0)
        return jnp.where(gid == g, jnp.maximum(pooled, loc[None, :]),
                         pooled)

    pooled_ref[...] = lax.fori_loop(lo_ref[i], hi_ref[i] + 1, body,
                                    pooled_ref[...])

    @pl.when(i == pl.num_programs(0) - 1)
    def _():
        p = pooled_ref[...]
        FW = fw_ref[...]
        FB = fb_ref[...]
        n = FW.shape[0]
        for l in range(n):
            p = jnp.dot(p, FW[l],
                        preferred_element_type=jnp.float32) + FB[l:l + 1, :]
            if l < n - 1:
                p = jnp.maximum(p, 0.0)
        out_ref[...] = p


def _post(h, agg, batch, nn_W1, nn_b1, nn_W2, nn_b2, fc2_W, fc2_b):
    n_tiles = h.shape[0] // ROW_TILE
    b1 = nn_b1.reshape(1, D)
    b2 = nn_b2.reshape(1, D)
    batch = batch.astype(jnp.int32)
    btiles = batch.reshape(n_tiles, ROW_TILE)
    lo = btiles[:, 0]
    hi = btiles[:, -1]
    seg = jnp.sum(batch[None, :] <
                  jnp.arange(NUM_GRAPHS + 1, dtype=jnp.int32)[:, None],
                  axis=1, dtype=jnp.int32)
    return pl.pallas_call(
        _post_body,
        grid=(n_tiles,),
        in_specs=[
            pl.BlockSpec(memory_space=pltpu.SMEM),
            pl.BlockSpec(memory_space=pltpu.SMEM),
            pl.BlockSpec(memory_space=pltpu.SMEM),
            pl.BlockSpec((ROW_TILE, D), lambda i: (i, 0)),
            pl.BlockSpec((2, ROW_TILE, D), lambda i: (0, i, 0)),
            pl.BlockSpec((D, D), lambda i: (0, 0)),
            pl.BlockSpec((1, D), lambda i: (0, 0)),
            pl.BlockSpec((D, D), lambda i: (0, 0)),
            pl.BlockSpec((1, D), lambda i: (0, 0)),
            pl.BlockSpec(fc2_W.shape, lambda i: (0, 0, 0)),
            pl.BlockSpec(fc2_b.shape, lambda i: (0, 0)),
        ],
        out_specs=pl.BlockSpec((NUM_GRAPHS, D), lambda i: (0, 0)),
        out_shape=jax.ShapeDtypeStruct((NUM_GRAPHS, D), jnp.float32),
        scratch_shapes=[pltpu.VMEM((NUM_GRAPHS, D), jnp.float32)],
    )(lo, hi, seg, h, agg, nn_W1, b1, nn_W2, b2, fc2_W, fc2_b)


def kernel(x, edge_index, batch, fc1_W, fc1_b, nn_W1, nn_b1, nn_W2, nn_b2,
           fc2_W, fc2_b):
    E = edge_index.shape[1]
    epw = -(-E // (NW * SCH * CH)) * SCH * CH
    e_pad = epw * NW
    pad = e_pad - E

    src = edge_index[0].astype(jnp.int32)
    dst = edge_index[1].astype(jnp.int32)
    if pad:
        ar = jnp.arange(pad, dtype=jnp.int32)
        src = jnp.concatenate([src, (ar * 97) % N_NODES])
        dst = jnp.concatenate([dst, N_NODES + ar % (ACC_ROWS - N_NODES)])
    src3d = src.reshape(NW * (epw // (SCH * CH)), SCH, CH)
    dst3d = dst.reshape(NW * (epw // (SCH * CH)), SCH, CH)
    h, r = _fc1(x, fc1_W, fc1_b)
    agg = _sc_agg(r, src3d, dst3d)
    return _post(h, agg, batch, nn_W1, nn_b1, nn_W2, nn_b2, fc2_W, fc2_b)


# speedup vs baseline: 1.0295x; 1.0030x over previous
import functools

import jax
import jax.numpy as jnp
from jax import lax
from jax.experimental import pallas as pl
from jax.experimental.pallas import tpu as pltpu
from jax.experimental.pallas import tpu_sc as plsc

N_NODES = 10000
D = 128
NUM_GRAPHS = 64
NEG_SLOPE = 0.01

ROW_TILE = 1000
CH = 80
SCH = 25
NW = 32
ACC_ROWS = 10240


def _leaky(x):
    return jnp.where(x >= 0, x, NEG_SLOPE * x)


def _fc1_body(x_ref, w_ref, b_ref, h_ref, r_ref):
    h = x_ref[...]
    W = w_ref[...]
    B = b_ref[...]
    n = W.shape[0]
    for l in range(n):
        h = jnp.dot(h, W[l], preferred_element_type=jnp.float32) + B[l:l + 1, :]
        if l < n - 1:
            h = jnp.maximum(h, 0.0)
    h = _leaky(h)
    h_ref[...] = h
    r_ref[...] = jnp.maximum(h, 0.0)


def _fc1(x, fc1_W, fc1_b):
    n_tiles = x.shape[0] // ROW_TILE
    return pl.pallas_call(
        _fc1_body,
        grid=(n_tiles,),
        in_specs=[
            pl.BlockSpec((ROW_TILE, D), lambda i: (i, 0)),
            pl.BlockSpec(fc1_W.shape, lambda i: (0, 0, 0)),
            pl.BlockSpec(fc1_b.shape, lambda i: (0, 0)),
        ],
        out_specs=[
            pl.BlockSpec((ROW_TILE, D), lambda i: (i, 0)),
            pl.BlockSpec((ROW_TILE, D), lambda i: (i, 0)),
        ],
        out_shape=[
            jax.ShapeDtypeStruct(x.shape, jnp.float32),
            jax.ShapeDtypeStruct(x.shape, jnp.float32),
        ],
    )(x, fc1_W, fc1_b)


def _sc_agg(r, src3d, dst3d):
    nsup = src3d.shape[0] // NW
    zc = ACC_ROWS // 16 // CH
    orows = ACC_ROWS // 16
    mesh = plsc.VectorSubcoreMesh(core_axis_name="c", subcore_axis_name="s")

    @functools.partial(
        pl.kernel,
        mesh=mesh,
        out_type=jax.ShapeDtypeStruct((2, ACC_ROWS, D), jnp.float32),
        scratch_types=[
            pltpu.VMEM((SCH, CH), jnp.int32),
            pltpu.VMEM((SCH, CH), jnp.int32),
            pltpu.VMEM((CH, D), jnp.float32),
            pltpu.VMEM((CH, D), jnp.float32),
            pltpu.VMEM((CH, D), jnp.float32),
            pltpu.VMEM((CH, D), jnp.float32),
            pltpu.VMEM_SHARED((ACC_ROWS, D), jnp.float32),
            pltpu.SemaphoreType.DMA,
            pltpu.SemaphoreType.DMA,
            pltpu.SemaphoreType.DMA,
            pltpu.SemaphoreType.DMA,
        ],
    )
    def k(r_hbm, src_hbm, dst_hbm, out_hbm, src_v, dst_v, rows0, rows1,
          rows2, rows3, acc_sh, sem0, sem1, sem2, sem3):
        cid = lax.axis_index("c")
        sid = lax.axis_index("s")
        wid = cid * 16 + sid
        rows = (rows0, rows1, rows2, rows3)
        sems = (sem0, sem1, sem2, sem3)

        @pl.loop(0, CH)
        def _(rr):
            @pl.loop(0, D, step=16)
            def _(cc):
                rows0.at[pl.ds(rr, 1), pl.ds(cc, 16)][...] = jnp.zeros(
                    (1, 16), jnp.float32)

        for kk in range(zc):
            pltpu.async_copy(rows0,
                             acc_sh.at[pl.ds(sid * (zc * CH) + kk * CH, CH)],
                             sems[kk % 4])
        for kk in range(zc):
            pltpu.make_async_copy(
                rows0, acc_sh.at[pl.ds(sid * (zc * CH) + kk * CH, CH)],
                sems[kk % 4]).wait()

        plsc.subcore_barrier()

        @pl.loop(0, nsup)
        def _(s):
            pltpu.sync_copy(src_hbm.at[wid * nsup + s], src_v)
            pltpu.sync_copy(dst_hbm.at[wid * nsup + s], dst_v)
            for p in range(3):
                pltpu.async_copy(r_hbm.at[src_v.at[p]], rows[p], sems[p])

            @pl.loop(0, SCH, step=4)
            def _(c):
                for kk in range(4):
                    slot = kk
                    pref = (kk + 3) % 4

                    def blk(kk=kk, slot=slot, pref=pref):
                        @pl.when(c + kk + 3 < SCH)
                        def _():
                            pltpu.async_copy(
                                r_hbm.at[src_v.at[c + kk + 3]],
                                rows[pref], sems[pref])

                        pltpu.make_async_copy(r_hbm.at[src_v.at[c + kk]],
                                              rows[slot], sems[slot]).wait()
                        pltpu.sync_copy(rows[slot],
                                        acc_sh.at[dst_v.at[c + kk]],
                                        add=True)

                    if kk == 0:
                        blk()
                    else:
                        pl.when(c + kk < SCH)(blk)

        plsc.subcore_barrier()

        pltpu.sync_copy(acc_sh.at[pl.ds(sid * orows, orows)],
                        out_hbm.at[cid, pl.ds(sid * orows, orows)])

    return k(r, src3d, dst3d)


def _post_body(lo_ref, hi_ref, seg_ref, h_ref, agg_ref, w1_ref, b1_ref,
               w2_ref, b2_ref, fw_ref, fb_ref, out_ref, pooled_ref):
    i = pl.program_id(0)
    z = h_ref[...] + agg_ref[0] + agg_ref[1]
    z = jnp.maximum(jnp.dot(z, w1_ref[...],
                            preferred_element_type=jnp.float32)
                    + b1_ref[...][0:1, :], 0.0)
    z = jnp.dot(z, w2_ref[...],
                preferred_element_type=jnp.float32) + b2_ref[...][0:1, :]
    z = _leaky(z)

    @pl.when(i == 0)
    def _():
        pooled_ref[...] = jnp.full((NUM_GRAPHS, D), -jnp.inf, jnp.float32)

    rowid = lax.broadcasted_iota(jnp.int32, (ROW_TILE, 1), 0) + i * ROW_TILE
    gid = lax.broadcasted_iota(jnp.int32, (NUM_GRAPHS, 1), 0)

    def body(g, pooled):
        m = (rowid >= seg_ref[g]) & (rowid < seg_ref[g + 1])
        loc = jnp.max(jnp.where(m, z, -jnp.inf), axis=0)
        return jnp.where(gid == g, jnp.maximum(pooled, loc[None, :]),
                         pooled)

    pooled_ref[...] = lax.fori_loop(lo_ref[i], hi_ref[i] + 1, body,
                                    pooled_ref[...])

    @pl.when(i == pl.num_programs(0) - 1)
    def _():
        p = pooled_ref[...]
        FW = fw_ref[...]
        FB = fb_ref[...]
        n = FW.shape[0]
        for l in range(n):
            p = jnp.dot(p, FW[l],
                        preferred_element_type=jnp.float32) + FB[l:l + 1, :]
            if l < n - 1:
                p = jnp.maximum(p, 0.0)
        out_ref[...] = p


def _post(h, agg, batch, nn_W1, nn_b1, nn_W2, nn_b2, fc2_W, fc2_b):
    n_tiles = h.shape[0] // ROW_TILE
    b1 = nn_b1.reshape(1, D)
    b2 = nn_b2.reshape(1, D)
    batch = batch.astype(jnp.int32)
    btiles = batch.reshape(n_tiles, ROW_TILE)
    lo = btiles[:, 0]
    hi = btiles[:, -1]
    seg = jnp.sum(batch[None, :] <
                  jnp.arange(NUM_GRAPHS + 1, dtype=jnp.int32)[:, None],
                  axis=1, dtype=jnp.int32)
    return pl.pallas_call(
        _post_body,
        grid=(n_tiles,),
        in_specs=[
            pl.BlockSpec(memory_space=pltpu.SMEM),
            pl.BlockSpec(memory_space=pltpu.SMEM),
            pl.BlockSpec(memory_space=pltpu.SMEM),
            pl.BlockSpec((ROW_TILE, D), lambda i: (i, 0)),
            pl.BlockSpec((2, ROW_TILE, D), lambda i: (0, i, 0)),
            pl.BlockSpec((D, D), lambda i: (0, 0)),
            pl.BlockSpec((1, D), lambda i: (0, 0)),
            pl.BlockSpec((D, D), lambda i: (0, 0)),
            pl.BlockSpec((1, D), lambda i: (0, 0)),
            pl.BlockSpec(fc2_W.shape, lambda i: (0, 0, 0)),
            pl.BlockSpec(fc2_b.shape, lambda i: (0, 0)),
        ],
        out_specs=pl.BlockSpec((NUM_GRAPHS, D), lambda i: (0, 0)),
        out_shape=jax.ShapeDtypeStruct((NUM_GRAPHS, D), jnp.float32),
        scratch_shapes=[pltpu.VMEM((NUM_GRAPHS, D), jnp.float32)],
    )(lo, hi, seg, h, agg, nn_W1, b1, nn_W2, b2, fc2_W, fc2_b)


def kernel(x, edge_index, batch, fc1_W, fc1_b, nn_W1, nn_b1, nn_W2, nn_b2,
           fc2_W, fc2_b):
    E = edge_index.shape[1]
    epw = -(-E // (NW * SCH * CH)) * SCH * CH
    e_pad = epw * NW
    pad = e_pad - E

    src = edge_index[0].astype(jnp.int32)
    dst = edge_index[1].astype(jnp.int32)
    if pad:
        ar = jnp.arange(pad, dtype=jnp.int32)
        src = jnp.concatenate([src, (ar * 97) % N_NODES])
        dst = jnp.concatenate([dst, N_NODES + ar % (ACC_ROWS - N_NODES)])
    src3d = src.reshape(NW * (epw // (SCH * CH)), SCH, CH)
    dst3d = dst.reshape(NW * (epw // (SCH * CH)), SCH, CH)
    h, r = _fc1(x, fc1_W, fc1_b)
    agg = _sc_agg(r, src3d, dst3d)
    return _post(h, agg, batch, nn_W1, nn_b1, nn_W2, nn_b2, fc2_W, fc2_b)


# speedup vs baseline: 1.0332x; 1.0037x over previous
import functools

import jax
import jax.numpy as jnp
from jax import lax
from jax.experimental import pallas as pl
from jax.experimental.pallas import tpu as pltpu
from jax.experimental.pallas import tpu_sc as plsc

N_NODES = 10000
D = 128
NUM_GRAPHS = 64
NEG_SLOPE = 0.01

ROW_TILE = 1000
CH = 80
SCH = 25
NW = 32
ACC_ROWS = 10240


def _leaky(x):
    return jnp.where(x >= 0, x, NEG_SLOPE * x)


def _fc1_body(x_ref, w_ref, b_ref, h_ref, r_ref):
    h = x_ref[...]
    W = w_ref[...]
    B = b_ref[...]
    n = W.shape[0]
    for l in range(n):
        h = jnp.dot(h, W[l], preferred_element_type=jnp.float32) + B[l:l + 1, :]
        if l < n - 1:
            h = jnp.maximum(h, 0.0)
    h = _leaky(h)
    h_ref[...] = h
    r_ref[...] = jnp.maximum(h, 0.0)


def _fc1(x, fc1_W, fc1_b):
    n_tiles = x.shape[0] // ROW_TILE
    return pl.pallas_call(
        _fc1_body,
        grid=(n_tiles,),
        in_specs=[
            pl.BlockSpec((ROW_TILE, D), lambda i: (i, 0)),
            pl.BlockSpec(fc1_W.shape, lambda i: (0, 0, 0)),
            pl.BlockSpec(fc1_b.shape, lambda i: (0, 0)),
        ],
        out_specs=[
            pl.BlockSpec((ROW_TILE, D), lambda i: (i, 0)),
            pl.BlockSpec((ROW_TILE, D), lambda i: (i, 0)),
        ],
        out_shape=[
            jax.ShapeDtypeStruct(x.shape, jnp.float32),
            jax.ShapeDtypeStruct(x.shape, jnp.float32),
        ],
    )(x, fc1_W, fc1_b)


def _sc_agg(r, src3d, dst3d):
    nsup = src3d.shape[0] // NW
    zc = ACC_ROWS // 16 // CH
    orows = ACC_ROWS // 16
    mesh = plsc.VectorSubcoreMesh(core_axis_name="c", subcore_axis_name="s")

    @functools.partial(
        pl.kernel,
        mesh=mesh,
        out_type=jax.ShapeDtypeStruct((2, ACC_ROWS, D), jnp.float32),
        scratch_types=[
            pltpu.VMEM((SCH, CH), jnp.int32),
            pltpu.VMEM((SCH, CH), jnp.int32),
            pltpu.VMEM((CH, D), jnp.float32),
            pltpu.VMEM((CH, D), jnp.float32),
            pltpu.VMEM((CH, D), jnp.float32),
            pltpu.VMEM((CH, D), jnp.float32),
            pltpu.VMEM_SHARED((ACC_ROWS, D), jnp.float32),
            pltpu.SemaphoreType.DMA,
            pltpu.SemaphoreType.DMA,
            pltpu.SemaphoreType.DMA,
            pltpu.SemaphoreType.DMA,
            pltpu.SemaphoreType.DMA,
            pltpu.SemaphoreType.DMA,
            pltpu.SemaphoreType.DMA,
            pltpu.SemaphoreType.DMA,
        ],
    )
    def k(r_hbm, src_hbm, dst_hbm, out_hbm, src_v, dst_v, rows0, rows1,
          rows2, rows3, acc_sh, sem0, sem1, sem2, sem3, ssem0, ssem1,
          ssem2, ssem3):
        cid = lax.axis_index("c")
        sid = lax.axis_index("s")
        wid = cid * 16 + sid
        rows = (rows0, rows1, rows2, rows3)
        sems = (sem0, sem1, sem2, sem3)
        ssems = (ssem0, ssem1, ssem2, ssem3)

        @pl.loop(0, CH)
        def _(rr):
            @pl.loop(0, D, step=16)
            def _(cc):
                rows0.at[pl.ds(rr, 1), pl.ds(cc, 16)][...] = jnp.zeros(
                    (1, 16), jnp.float32)

        for kk in range(zc):
            pltpu.async_copy(rows0,
                             acc_sh.at[pl.ds(sid * (zc * CH) + kk * CH, CH)],
                             sems[kk % 4])
        for kk in range(zc):
            pltpu.make_async_copy(
                rows0, acc_sh.at[pl.ds(sid * (zc * CH) + kk * CH, CH)],
                sems[kk % 4]).wait()

        plsc.subcore_barrier()

        @pl.loop(0, nsup)
        def _(s):
            pltpu.sync_copy(src_hbm.at[wid * nsup + s], src_v)
            pltpu.sync_copy(dst_hbm.at[wid * nsup + s], dst_v)
            for p in range(3):
                pltpu.async_copy(r_hbm.at[src_v.at[p]], rows[p], sems[p])

            @pl.loop(0, SCH, step=4)
            def _(c):
                for kk in range(4):
                    slot = kk
                    pref = (kk + 3) % 4

                    def blk(kk=kk, slot=slot, pref=pref):
                        @pl.when(c + kk + 3 < SCH)
                        def _():
                            def w():
                                pltpu.make_async_copy(
                                    rows[pref],
                                    acc_sh.at[dst_v.at[c + kk - 1]],
                                    ssems[pref]).wait()

                            if kk == 0:
                                pl.when(c >= 1)(w)
                            else:
                                w()
                            pltpu.async_copy(
                                r_hbm.at[src_v.at[c + kk + 3]],
                                rows[pref], sems[pref])

                        pltpu.make_async_copy(r_hbm.at[src_v.at[c + kk]],
                                              rows[slot], sems[slot]).wait()
                        pltpu.async_copy(rows[slot],
                                         acc_sh.at[dst_v.at[c + kk]],
                                         ssems[slot], add=True)

                    if kk == 0:
                        blk()
                    else:
                        pl.when(c + kk < SCH)(blk)

            for t in range(4):
                m = SCH - 4 + t
                pltpu.make_async_copy(rows[m % 4],
                                      acc_sh.at[dst_v.at[m]],
                                      ssems[m % 4]).wait()

        plsc.subcore_barrier()

        pltpu.sync_copy(acc_sh.at[pl.ds(sid * orows, orows)],
                        out_hbm.at[cid, pl.ds(sid * orows, orows)])

    return k(r, src3d, dst3d)


def _post_body(lo_ref, hi_ref, seg_ref, h_ref, agg_ref, w1_ref, b1_ref,
               w2_ref, b2_ref, fw_ref, fb_ref, out_ref, pooled_ref):
    i = pl.program_id(0)
    z = h_ref[...] + agg_ref[0] + agg_ref[1]
    z = jnp.maximum(jnp.dot(z, w1_ref[...],
                            preferred_element_type=jnp.float32)
                    + b1_ref[...][0:1, :], 0.0)
    z = jnp.dot(z, w2_ref[...],
                preferred_element_type=jnp.float32) + b2_ref[...][0:1, :]
    z = _leaky(z)

    @pl.when(i == 0)
    def _():
        pooled_ref[...] = jnp.full((NUM_GRAPHS, D), -jnp.inf, jnp.float32)

    rowid = lax.broadcasted_iota(jnp.int32, (ROW_TILE, 1), 0) + i * ROW_TILE
    gid = lax.broadcasted_iota(jnp.int32, (NUM_GRAPHS, 1), 0)

    def body(g, pooled):
        m = (rowid >= seg_ref[g]) & (rowid < seg_ref[g + 1])
        loc = jnp.max(jnp.where(m, z, -jnp.inf), axis=0)
        return jnp.where(gid == g, jnp.maximum(pooled, loc[None, :]),
                         pooled)

    pooled_ref[...] = lax.fori_loop(lo_ref[i], hi_ref[i] + 1, body,
                                    pooled_ref[...])

    @pl.when(i == pl.num_programs(0) - 1)
    def _():
        p = pooled_ref[...]
        FW = fw_ref[...]
        FB = fb_ref[...]
        n = FW.shape[0]
        for l in range(n):
            p = jnp.dot(p, FW[l],
                        preferred_element_type=jnp.float32) + FB[l:l + 1, :]
            if l < n - 1:
                p = jnp.maximum(p, 0.0)
        out_ref[...] = p


def _post(h, agg, batch, nn_W1, nn_b1, nn_W2, nn_b2, fc2_W, fc2_b):
    n_tiles = h.shape[0] // ROW_TILE
    b1 = nn_b1.reshape(1, D)
    b2 = nn_b2.reshape(1, D)
    batch = batch.astype(jnp.int32)
    btiles = batch.reshape(n_tiles, ROW_TILE)
    lo = btiles[:, 0]
    hi = btiles[:, -1]
    seg = jnp.sum(batch[None, :] <
                  jnp.arange(NUM_GRAPHS + 1, dtype=jnp.int32)[:, None],
                  axis=1, dtype=jnp.int32)
    return pl.pallas_call(
        _post_body,
        grid=(n_tiles,),
        in_specs=[
            pl.BlockSpec(memory_space=pltpu.SMEM),
            pl.BlockSpec(memory_space=pltpu.SMEM),
            pl.BlockSpec(memory_space=pltpu.SMEM),
            pl.BlockSpec((ROW_TILE, D), lambda i: (i, 0)),
            pl.BlockSpec((2, ROW_TILE, D), lambda i: (0, i, 0)),
            pl.BlockSpec((D, D), lambda i: (0, 0)),
            pl.BlockSpec((1, D), lambda i: (0, 0)),
            pl.BlockSpec((D, D), lambda i: (0, 0)),
            pl.BlockSpec((1, D), lambda i: (0, 0)),
            pl.BlockSpec(fc2_W.shape, lambda i: (0, 0, 0)),
            pl.BlockSpec(fc2_b.shape, lambda i: (0, 0)),
        ],
        out_specs=pl.BlockSpec((NUM_GRAPHS, D), lambda i: (0, 0)),
        out_shape=jax.ShapeDtypeStruct((NUM_GRAPHS, D), jnp.float32),
        scratch_shapes=[pltpu.VMEM((NUM_GRAPHS, D), jnp.float32)],
    )(lo, hi, seg, h, agg, nn_W1, b1, nn_W2, b2, fc2_W, fc2_b)


def kernel(x, edge_index, batch, fc1_W, fc1_b, nn_W1, nn_b1, nn_W2, nn_b2,
           fc2_W, fc2_b):
    E = edge_index.shape[1]
    epw = -(-E // (NW * SCH * CH)) * SCH * CH
    e_pad = epw * NW
    pad = e_pad - E

    src = edge_index[0].astype(jnp.int32)
    dst = edge_index[1].astype(jnp.int32)
    if pad:
        ar = jnp.arange(pad, dtype=jnp.int32)
        src = jnp.concatenate([src, (ar * 97) % N_NODES])
        dst = jnp.concatenate([dst, N_NODES + ar % (ACC_ROWS - N_NODES)])
    src3d = src.reshape(NW * (epw // (SCH * CH)), SCH, CH)
    dst3d = dst.reshape(NW * (epw // (SCH * CH)), SCH, CH)
    h, r = _fc1(x, fc1_W, fc1_b)
    agg = _sc_agg(r, src3d, dst3d)
    return _post(h, agg, batch, nn_W1, nn_b1, nn_W2, nn_b2, fc2_W, fc2_b)
